# Initial kernel scaffold; baseline (speedup 1.0000x reference)
import jax
import jax.numpy as jnp
from jax.experimental import pallas as pl


def kernel(s, z, W0, b0, W1, b1, W2, b2, Wf, bf, gamma, beta, edge_index):
    raise NotImplementedError("write your pallas kernel here")


# speedup vs baseline: 2.3830x; 2.3830x over previous
import functools

import jax
import jax.numpy as jnp
from jax import lax
from jax.experimental import pallas as pl
from jax.experimental.pallas import tpu as pltpu
from jax.experimental.pallas import tpu_sc as plsc

N_NODES = 10000
N_EDGES = 160000
C_N = 256
C_Z = 128
BIAS = 128
HID = 384


def _embed_body(s_ref, w0_ref, b0_ref, o_ref):
    o_ref[...] = (
        jnp.dot(s_ref[...], w0_ref[...], preferred_element_type=jnp.float32)
        + b0_ref[...]
    )


def _node_embed(s, W0, b0):
    TN = 2000
    return pl.pallas_call(
        _embed_body,
        grid=(N_NODES // TN,),
        in_specs=[
            pl.BlockSpec((TN, C_N), lambda i: (i, 0)),
            pl.BlockSpec((C_N, BIAS), lambda i: (0, 0)),
            pl.BlockSpec((1, BIAS), lambda i: (0, 0)),
        ],
        out_specs=pl.BlockSpec((TN, BIAS), lambda i: (i, 0)),
        out_shape=jax.ShapeDtypeStruct((N_NODES, BIAS), jnp.float32),
    )(s, W0, b0.reshape(1, BIAS))


def _gather_sc(node_emb, idx_flat):
    info = plsc.get_sparse_core_info()
    nw = info.num_cores * info.num_subcores
    ch = 128
    n_rows = 2 * N_EDGES
    n_chunks = n_rows // ch
    cpw = (n_chunks + nw - 1) // nw
    mesh = plsc.VectorSubcoreMesh(core_axis_name="c", subcore_axis_name="s")

    @functools.partial(
        pl.kernel,
        mesh=mesh,
        out_type=jax.ShapeDtypeStruct((n_rows, BIAS), jnp.float32),
        scratch_types=[
            pltpu.VMEM((ch,), jnp.int32),
            pltpu.VMEM((ch, BIAS), jnp.float32),
            pltpu.SemaphoreType.DMA,
        ],
    )
    def gk(node_hbm, idx_hbm, out_hbm, idx_v, rows_v, sem):
        w = lax.axis_index("s") * info.num_cores + lax.axis_index("c")

        def body(t, carry):
            cid = w * cpw + t

            @pl.when(cid < n_chunks)
            def _():
                off = cid * ch
                pltpu.sync_copy(idx_hbm.at[pl.ds(off, ch)], idx_v)
                pltpu.async_copy(node_hbm.at[idx_v], rows_v, sem).wait()
                pltpu.sync_copy(rows_v, out_hbm.at[pl.ds(off, ch)])

            return carry

        lax.fori_loop(0, cpw, body, 0)

    return gk(node_emb, idx_flat)


def _mlp_body(z_ref, s_ref, d_ref, w1_ref, b1_ref, w2_ref, b2_ref,
              wf_ref, bf_ref, g_ref, be_ref, o_ref):
    e = jnp.concatenate([z_ref[...], s_ref[...], d_ref[...]], axis=1)
    h = jnp.maximum(
        jnp.dot(e, w1_ref[...], preferred_element_type=jnp.float32)
        + b1_ref[...], 0.0)
    h = jnp.maximum(
        jnp.dot(h, w2_ref[...], preferred_element_type=jnp.float32)
        + b2_ref[...], 0.0)
    o = (jnp.dot(h + e, wf_ref[...], preferred_element_type=jnp.float32)
         + bf_ref[...])
    mu = jnp.mean(o, axis=1, keepdims=True)
    c = o - mu
    var = jnp.mean(c * c, axis=1, keepdims=True)
    o_ref[...] = c * lax.rsqrt(var + 1e-5) * g_ref[...] + be_ref[...]


def _edge_mlp(z, gathered, W1, b1, W2, b2, Wf, bf, gamma, beta):
    TE = 800
    G = N_EDGES // TE
    return pl.pallas_call(
        _mlp_body,
        grid=(G,),
        in_specs=[
            pl.BlockSpec((TE, C_Z), lambda i: (i, 0)),
            pl.BlockSpec((TE, BIAS), lambda i: (i, 0)),
            pl.BlockSpec((TE, BIAS), lambda i: (i + G, 0)),
            pl.BlockSpec((HID, HID), lambda i: (0, 0)),
            pl.BlockSpec((1, HID), lambda i: (0, 0)),
            pl.BlockSpec((HID, HID), lambda i: (0, 0)),
            pl.BlockSpec((1, HID), lambda i: (0, 0)),
            pl.BlockSpec((HID, C_Z), lambda i: (0, 0)),
            pl.BlockSpec((1, C_Z), lambda i: (0, 0)),
            pl.BlockSpec((1, C_Z), lambda i: (0, 0)),
            pl.BlockSpec((1, C_Z), lambda i: (0, 0)),
        ],
        out_specs=pl.BlockSpec((TE, C_Z), lambda i: (i, 0)),
        out_shape=jax.ShapeDtypeStruct((N_EDGES, C_Z), jnp.float32),
    )(z, gathered, gathered, W1, b1.reshape(1, HID), W2,
      b2.reshape(1, HID), Wf, bf.reshape(1, C_Z), gamma.reshape(1, C_Z),
      beta.reshape(1, C_Z))


def kernel(s, z, W0, b0, W1, b1, W2, b2, Wf, bf, gamma, beta, edge_index):
    node_emb = _node_embed(s, W0, b0)
    gathered = _gather_sc(node_emb, edge_index.reshape(-1))
    return _edge_mlp(z, gathered, W1, b1, W2, b2, Wf, bf, gamma, beta)


# speedup vs baseline: 3.1509x; 1.3222x over previous
import functools

import jax
import jax.numpy as jnp
from jax import lax
from jax.experimental import pallas as pl
from jax.experimental.pallas import tpu as pltpu
from jax.experimental.pallas import tpu_sc as plsc

N_NODES = 10000
N_EDGES = 160000
C_N = 256
C_Z = 128
BIAS = 128
HID = 384

P = 5
EP = N_EDGES // P
TE = 800
GP = EP // TE
CH = 128


def _embed_body(s_ref, w0_ref, b0_ref, o_ref):
    o_ref[...] = (
        jnp.dot(s_ref[...], w0_ref[...], preferred_element_type=jnp.float32)
        + b0_ref[...])


def _node_embed(s, W0, b0):
    TN = 2000
    return pl.pallas_call(
        _embed_body,
        grid=(N_NODES // TN,),
        in_specs=[
            pl.BlockSpec((TN, C_N), lambda i: (i, 0)),
            pl.BlockSpec((C_N, BIAS), lambda i: (0, 0)),
            pl.BlockSpec((1, BIAS), lambda i: (0, 0)),
        ],
        out_specs=pl.BlockSpec((TN, BIAS), lambda i: (i, 0)),
        out_shape=jax.ShapeDtypeStruct((N_NODES, BIAS), jnp.float32),
    )(s, W0, b0.reshape(1, BIAS))


def _gather_part(node_emb, idx_flat, part):
    info = plsc.get_sparse_core_info()
    nw = info.num_cores * info.num_subcores
    half = EP // CH
    n_chunks = 2 * half
    cpw = (n_chunks + nw - 1) // nw
    src_base = part * EP
    dst_base = N_EDGES + part * EP
    mesh = plsc.VectorSubcoreMesh(core_axis_name="c", subcore_axis_name="s")

    @functools.partial(
        pl.kernel,
        mesh=mesh,
        out_type=jax.ShapeDtypeStruct((2 * EP, BIAS), jnp.float32),
        scratch_types=[
            pltpu.VMEM((CH,), jnp.int32),
            pltpu.VMEM((CH, BIAS), jnp.float32),
            pltpu.SemaphoreType.DMA,
        ],
    )
    def gk(node_hbm, idx_hbm, out_hbm, idx_v, rows_v, sem):
        w = lax.axis_index("s") * info.num_cores + lax.axis_index("c")

        def body(t, carry):
            cid = w * cpw + t

            @pl.when(cid < n_chunks)
            def _():
                out_off = cid * CH
                idx_off = jnp.where(cid < half,
                                    src_base + cid * CH,
                                    dst_base + (cid - half) * CH)
                pltpu.sync_copy(idx_hbm.at[pl.ds(idx_off, CH)], idx_v)
                pltpu.async_copy(node_hbm.at[idx_v], rows_v, sem).wait()
                pltpu.sync_copy(rows_v, out_hbm.at[pl.ds(out_off, CH)])

            return carry

        lax.fori_loop(0, cpw, body, 0)

    return gk(node_emb, idx_flat)


def _mlp_body(z_ref, s_ref, d_ref, w1_ref, b1_ref, w2_ref, b2_ref,
              wf_ref, bf_ref, g_ref, be_ref, prev_ref, o_ref):
    del prev_ref
    e = jnp.concatenate([z_ref[...], s_ref[...], d_ref[...]], axis=1)
    h = jnp.maximum(
        jnp.dot(e, w1_ref[...], preferred_element_type=jnp.float32)
        + b1_ref[...], 0.0)
    h = jnp.maximum(
        jnp.dot(h, w2_ref[...], preferred_element_type=jnp.float32)
        + b2_ref[...], 0.0)
    o = (jnp.dot(h + e, wf_ref[...], preferred_element_type=jnp.float32)
         + bf_ref[...])
    mu = jnp.mean(o, axis=1, keepdims=True)
    c = o - mu
    var = jnp.mean(c * c, axis=1, keepdims=True)
    o_ref[...] = c * lax.rsqrt(var + 1e-5) * g_ref[...] + be_ref[...]


def _mlp_body_first(z_ref, s_ref, d_ref, w1_ref, b1_ref, w2_ref, b2_ref,
                    wf_ref, bf_ref, g_ref, be_ref, o_ref):
    _mlp_body(z_ref, s_ref, d_ref, w1_ref, b1_ref, w2_ref, b2_ref,
              wf_ref, bf_ref, g_ref, be_ref, None, o_ref)


def _mlp_part(z, gathered, weights, prev_out, part):
    W1, b1, W2, b2, Wf, bf, gamma, beta = weights

    def _const2(shape):
        return pl.BlockSpec(shape, lambda i: (0, 0))

    in_specs = [
        pl.BlockSpec((TE, C_Z), lambda i: (i + part * GP, 0)),
        pl.BlockSpec((TE, BIAS), lambda i: (i, 0)),
        pl.BlockSpec((TE, BIAS), lambda i: (i + GP, 0)),
        _const2((HID, HID)),
        _const2((1, HID)),
        _const2((HID, HID)),
        _const2((1, HID)),
        _const2((HID, C_Z)),
        _const2((1, C_Z)),
        _const2((1, C_Z)),
        _const2((1, C_Z)),
    ]
    args = [z, gathered, gathered, W1, b1.reshape(1, HID), W2,
            b2.reshape(1, HID), Wf, bf.reshape(1, C_Z),
            gamma.reshape(1, C_Z), beta.reshape(1, C_Z)]
    if prev_out is None:
        body = _mlp_body_first
        aliases = {}
    else:
        body = _mlp_body
        in_specs.append(pl.BlockSpec(memory_space=pl.ANY))
        args.append(prev_out)
        aliases = {11: 0}

    return pl.pallas_call(
        body,
        grid=(GP,),
        in_specs=in_specs,
        out_specs=pl.BlockSpec((TE, C_Z), lambda i: (i + part * GP, 0)),
        out_shape=jax.ShapeDtypeStruct((N_EDGES, C_Z), jnp.float32),
        input_output_aliases=aliases,
    )(*args)


def kernel(s, z, W0, b0, W1, b1, W2, b2, Wf, bf, gamma, beta, edge_index):
    node_emb = _node_embed(s, W0, b0)
    idx_flat = edge_index.reshape(-1)
    weights = (W1, b1, W2, b2, Wf, bf, gamma, beta)

    gathered = [_gather_part(node_emb, idx_flat, p) for p in range(P)]
    out = None
    for p in range(P):
        out = _mlp_part(z, gathered[p], weights, out, p)
    return out


# speedup vs baseline: 3.5784x; 1.1357x over previous
import functools

import jax
import jax.numpy as jnp
from jax import lax
from jax.experimental import pallas as pl
from jax.experimental.pallas import tpu as pltpu
from jax.experimental.pallas import tpu_sc as plsc

N_NODES = 10000
N_EDGES = 160000
C_N = 256
C_Z = 128
BIAS = 128
HID = 384

P = 5
EP = N_EDGES // P
TE = 1600
GP = EP // TE
CH = 128


def _embed_body(s_ref, w0_ref, b0_ref, o_ref):
    o_ref[...] = (
        jnp.dot(s_ref[...], w0_ref[...], preferred_element_type=jnp.float32)
        + b0_ref[...])


def _node_embed(s, W0, b0):
    TN = 2000
    return pl.pallas_call(
        _embed_body,
        grid=(N_NODES // TN,),
        in_specs=[
            pl.BlockSpec((TN, C_N), lambda i: (i, 0)),
            pl.BlockSpec((C_N, BIAS), lambda i: (0, 0)),
            pl.BlockSpec((1, BIAS), lambda i: (0, 0)),
        ],
        out_specs=pl.BlockSpec((TN, BIAS), lambda i: (i, 0)),
        out_shape=jax.ShapeDtypeStruct((N_NODES, BIAS), jnp.float32),
    )(s, W0, b0.reshape(1, BIAS))


def _gather_part(node_emb, idx_flat, part):
    info = plsc.get_sparse_core_info()
    nw = info.num_cores * info.num_subcores
    half = EP // CH
    n_chunks = 2 * half
    cpw = (n_chunks + nw - 1) // nw
    src_base = part * EP
    dst_base = N_EDGES + part * EP
    mesh = plsc.VectorSubcoreMesh(core_axis_name="c", subcore_axis_name="s")

    @functools.partial(
        pl.kernel,
        mesh=mesh,
        out_type=jax.ShapeDtypeStruct((2 * EP, BIAS), jnp.float32),
        scratch_types=[
            pltpu.VMEM((CH,), jnp.int32),
            pltpu.VMEM((CH, BIAS), jnp.float32),
            pltpu.SemaphoreType.DMA,
        ],
    )
    def gk(node_hbm, idx_hbm, out_hbm, idx_v, rows_v, sem):
        w = lax.axis_index("s") * info.num_cores + lax.axis_index("c")

        def body(t, carry):
            cid = w * cpw + t

            @pl.when(cid < n_chunks)
            def _():
                out_off = cid * CH
                idx_off = jnp.where(cid < half,
                                    src_base + cid * CH,
                                    dst_base + (cid - half) * CH)
                pltpu.sync_copy(idx_hbm.at[pl.ds(idx_off, CH)], idx_v)
                pltpu.async_copy(node_hbm.at[idx_v], rows_v, sem).wait()
                pltpu.sync_copy(rows_v, out_hbm.at[pl.ds(out_off, CH)])

            return carry

        lax.fori_loop(0, cpw, body, 0)

    return gk(node_emb, idx_flat)


def _mlp_body(z_ref, s_ref, d_ref, w1_ref, b1_ref, w2_ref, b2_ref,
              wf_ref, bf_ref, g_ref, be_ref, prev_ref, o_ref):
    del prev_ref
    e = jnp.concatenate([z_ref[...], s_ref[...], d_ref[...]], axis=1)
    h = jnp.maximum(
        jnp.dot(e, w1_ref[...], preferred_element_type=jnp.float32)
        + b1_ref[...], 0.0)
    h = jnp.maximum(
        jnp.dot(h, w2_ref[...], preferred_element_type=jnp.float32)
        + b2_ref[...], 0.0)
    o = (jnp.dot(h + e, wf_ref[...], preferred_element_type=jnp.float32)
         + bf_ref[...])
    mu = jnp.mean(o, axis=1, keepdims=True)
    c = o - mu
    var = jnp.mean(c * c, axis=1, keepdims=True)
    o_ref[...] = c * lax.rsqrt(var + 1e-5) * g_ref[...] + be_ref[...]


def _mlp_body_first(z_ref, s_ref, d_ref, w1_ref, b1_ref, w2_ref, b2_ref,
                    wf_ref, bf_ref, g_ref, be_ref, o_ref):
    _mlp_body(z_ref, s_ref, d_ref, w1_ref, b1_ref, w2_ref, b2_ref,
              wf_ref, bf_ref, g_ref, be_ref, None, o_ref)


def _mlp_part(z, gathered, weights, prev_out, part):
    W1, b1, W2, b2, Wf, bf, gamma, beta = weights

    def _const2(shape):
        return pl.BlockSpec(shape, lambda i: (0, 0))

    in_specs = [
        pl.BlockSpec((TE, C_Z), lambda i: (i + part * GP, 0)),
        pl.BlockSpec((TE, BIAS), lambda i: (i, 0)),
        pl.BlockSpec((TE, BIAS), lambda i: (i + GP, 0)),
        _const2((HID, HID)),
        _const2((1, HID)),
        _const2((HID, HID)),
        _const2((1, HID)),
        _const2((HID, C_Z)),
        _const2((1, C_Z)),
        _const2((1, C_Z)),
        _const2((1, C_Z)),
    ]
    args = [z, gathered, gathered, W1, b1.reshape(1, HID), W2,
            b2.reshape(1, HID), Wf, bf.reshape(1, C_Z),
            gamma.reshape(1, C_Z), beta.reshape(1, C_Z)]
    if prev_out is None:
        body = _mlp_body_first
        aliases = {}
    else:
        body = _mlp_body
        in_specs.append(pl.BlockSpec(memory_space=pl.ANY))
        args.append(prev_out)
        aliases = {11: 0}

    return pl.pallas_call(
        body,
        grid=(GP,),
        in_specs=in_specs,
        out_specs=pl.BlockSpec((TE, C_Z), lambda i: (i + part * GP, 0)),
        out_shape=jax.ShapeDtypeStruct((N_EDGES, C_Z), jnp.float32),
        input_output_aliases=aliases,
    )(*args)


def kernel(s, z, W0, b0, W1, b1, W2, b2, Wf, bf, gamma, beta, edge_index):
    node_emb = _node_embed(s, W0, b0)
    idx_flat = edge_index.reshape(-1)
    weights = (W1, b1, W2, b2, Wf, bf, gamma, beta)

    gathered = [_gather_part(node_emb, idx_flat, p) for p in range(P)]
    out = None
    for p in range(P):
        out = _mlp_part(z, gathered[p], weights, out, p)
    return out
